# baseline (speedup 1.0000x reference)
import functools

import jax
import jax.numpy as jnp
from jax import lax
from jax.experimental import pallas as pl
from jax.experimental.pallas import tpu as pltpu
from jax.experimental.pallas import tpu_sc as plsc

_CUTOFF = 5.0
_NUM_BASIS = 16
_BC = 64000

_NC = 2
_NS = 16
_NW = _NC * _NS

_PI = 3.14159265358979
_INV_C = 1.0 / _CUTOFF
_SQ2C = 0.6324555320336759
_P0 = 9.99997486e-01
_P1 = -1.66651677e-01
_P2 = 8.30951228e-03
_P3 = -1.84470858e-04


def _bessel_body(e_ref, o_ref):
    e = e_ref[...]
    r2 = jnp.sum(e * e, axis=0, keepdims=True)
    irs = lax.rsqrt(r2)
    scale = _SQ2C * irs
    thpi = _INV_C * (r2 * irs)
    n_col = (lax.broadcasted_iota(jnp.int32, (_NUM_BASIS, 1), 0) + 1
             ).astype(jnp.float32)
    m = lax.dot_general(n_col, thpi, (((1,), (0,)), ((), ())),
                        precision=lax.Precision.HIGHEST,
                        preferred_element_type=jnp.float32)
    k = jnp.round(m)
    u = (m - k) * _PI
    s = u * u
    p = _P3
    for c in (_P2, _P1, _P0):
        p = p * s + c
    p = p * u
    sb = k.astype(jnp.int32) << 31
    y = lax.bitcast_convert_type(
        lax.bitcast_convert_type(p, jnp.int32) ^ sb, jnp.float32)
    o_ref[...] = y * scale


def _bessel_edges_t(ea_t):
    ne = ea_t.shape[1]
    grid = ne // _BC
    assert grid * _BC == ne
    return pl.pallas_call(
        _bessel_body,
        grid=(grid,),
        in_specs=[pl.BlockSpec((3, _BC), lambda i: (0, i))],
        out_specs=pl.BlockSpec((_NUM_BASIS, _BC), lambda i: (0, i)),
        out_shape=jax.ShapeDtypeStruct((_NUM_BASIS, ne), jnp.float32),
    )(ea_t)


def _make_sc_gather(n, d, species):
    n_pad = ((n + 127) // 128) * 128
    nw = _NS
    b_full = ((-(-n_pad // nw) + 127) // 128) * 128
    b_last = n_pad - (nw - 1) * b_full
    r_last = n - (nw - 1) * b_full
    assert 0 < b_last <= b_full and b_last % 128 == 0
    assert 0 < r_last <= b_last and r_last % 8 == 0 and (b_last - r_last) % 16 == 0
    mesh = plsc.VectorSubcoreMesh(core_axis_name="c", subcore_axis_name="s",
                                  num_cores=1)

    @functools.partial(
        pl.kernel,
        mesh=mesh,
        compiler_params=pltpu.CompilerParams(use_tc_tiling_on_sc=True,
                                            needs_layout_passes=False),
        out_type=(jax.ShapeDtypeStruct((d, n_pad), jnp.float32),
                  jax.ShapeDtypeStruct((d, n_pad), jnp.float32)),
        scratch_types=[
            pltpu.VMEM((d, species), jnp.float32),
            pltpu.VMEM((d, species), jnp.float32),
            pltpu.VMEM((b_full,), jnp.int32),
            pltpu.VMEM((d, b_full), jnp.float32),
        ],
    )
    def gather(wxt_hbm, wzt_hbm, idx_hbm, ox_hbm, oz_hbm,
               wx_v, wz_v, idx_v, out_v):
        wid = lax.axis_index("s")
        base = wid * b_full
        pltpu.sync_copy(wxt_hbm, wx_v)
        pltpu.sync_copy(wzt_hbm, wz_v)

        def run(n_idx, size):
            pltpu.sync_copy(idx_hbm.at[pl.ds(base, n_idx)],
                            idx_v.at[pl.ds(0, n_idx)])
            if n_idx < size:
                zeros = jnp.zeros((16,), jnp.int32)
                for off in range(n_idx, size, 16):
                    idx_v[pl.ds(off, 16)] = zeros

            def table_pass(w_v, o_hbm):
                def body(ci, _):
                    off = ci * 16
                    idx = idx_v[pl.ds(off, 16)]
                    for f in range(d):
                        fvec = jnp.full((16,), f, jnp.int32)
                        vals = plsc.load_gather(w_v, [fvec, idx])
                        out_v[f, pl.ds(off, 16)] = vals
                    return 0

                lax.fori_loop(0, size // 16, body, 0)
                pltpu.sync_copy(out_v.at[:, pl.ds(0, size)],
                                o_hbm.at[:, pl.ds(base, size)])

            table_pass(wx_v, ox_hbm)
            table_pass(wz_v, oz_hbm)

        @pl.when(wid < nw - 1)
        def _full():
            run(b_full, b_full)

        @pl.when(wid == nw - 1)
        def _last():
            run(r_last, b_last)

    return gather


def kernel(x, edge_attr, W_x, W_z):
    n = x.shape[0]
    d = W_x.shape[1]
    gx, gz = _make_sc_gather(n, d, W_x.shape[0])(W_x.T, W_z.T, x)
    he_t = _bessel_edges_t(edge_attr.T)
    return gx[:, :n].T, gz[:, :n].T, he_t.T


# speedup vs baseline: 1.6804x; 1.6804x over previous
import functools

import jax
import jax.numpy as jnp
from jax import lax
from jax.experimental import pallas as pl
from jax.experimental.pallas import tpu as pltpu
from jax.experimental.pallas import tpu_sc as plsc

_CUTOFF = 5.0
_NUM_BASIS = 16
_BC = 64000

_NC = 2
_NS = 16
_NW = _NC * _NS

_PI = 3.14159265358979
_INV_C = 1.0 / _CUTOFF
_SQ2C = 0.6324555320336759
_P0 = 9.99997486e-01
_P1 = -1.66651677e-01
_P2 = 8.30951228e-03
_P3 = -1.84470858e-04


def _bessel_body(e_ref, o_ref):
    e = e_ref[...]
    r2 = jnp.sum(e * e, axis=0, keepdims=True)
    irs = lax.rsqrt(r2)
    scale = _SQ2C * irs
    thpi = _INV_C * (r2 * irs)
    n = (lax.broadcasted_iota(jnp.int32, (_NUM_BASIS, e.shape[1]), 0) + 1
         ).astype(jnp.float32)
    m = n * thpi
    k = jnp.round(m)
    u = (m - k) * _PI
    s = u * u
    p = _P3
    for c in (_P2, _P1, _P0):
        p = p * s + c
    p = p * u
    sb = k.astype(jnp.int32) << 31
    y = lax.bitcast_convert_type(
        lax.bitcast_convert_type(p, jnp.int32) ^ sb, jnp.float32)
    o_ref[...] = y * scale


def _bessel_edges_t(ea_t):
    ne = ea_t.shape[1]
    grid = ne // _BC
    assert grid * _BC == ne
    return pl.pallas_call(
        _bessel_body,
        grid=(grid,),
        in_specs=[pl.BlockSpec((3, _BC), lambda i: (0, i))],
        out_specs=pl.BlockSpec((_NUM_BASIS, _BC), lambda i: (0, i)),
        out_shape=jax.ShapeDtypeStruct((_NUM_BASIS, ne), jnp.float32),
    )(ea_t)


def _make_sc_gather(n, d, species):
    n_pad = ((n + 127) // 128) * 128
    nw = _NS
    b_full = ((-(-n_pad // nw) + 127) // 128) * 128
    b_last = n_pad - (nw - 1) * b_full
    r_last = n - (nw - 1) * b_full
    assert 0 < b_last <= b_full and b_last % 128 == 0
    assert 0 < r_last <= b_last and r_last % 8 == 0 and (b_last - r_last) % 16 == 0
    mesh = plsc.VectorSubcoreMesh(core_axis_name="c", subcore_axis_name="s",
                                  num_cores=1)

    @functools.partial(
        pl.kernel,
        mesh=mesh,
        compiler_params=pltpu.CompilerParams(use_tc_tiling_on_sc=True,
                                            needs_layout_passes=False),
        out_type=(jax.ShapeDtypeStruct((d, n_pad), jnp.float32),
                  jax.ShapeDtypeStruct((d, n_pad), jnp.float32)),
        scratch_types=[
            pltpu.VMEM((d, species), jnp.float32),
            pltpu.VMEM((d, species), jnp.float32),
            pltpu.VMEM((b_full,), jnp.int32),
            pltpu.VMEM((d, b_full), jnp.float32),
        ],
    )
    def gather(wxt_hbm, wzt_hbm, idx_hbm, ox_hbm, oz_hbm,
               wx_v, wz_v, idx_v, out_v):
        wid = lax.axis_index("s")
        base = wid * b_full
        pltpu.sync_copy(wxt_hbm, wx_v)
        pltpu.sync_copy(wzt_hbm, wz_v)

        def run(n_idx, size):
            pltpu.sync_copy(idx_hbm.at[pl.ds(base, n_idx)],
                            idx_v.at[pl.ds(0, n_idx)])
            if n_idx < size:
                zeros = jnp.zeros((16,), jnp.int32)
                for off in range(n_idx, size, 16):
                    idx_v[pl.ds(off, 16)] = zeros

            def table_pass(w_v, o_hbm):
                def body(ci, _):
                    off = ci * 16
                    idx = idx_v[pl.ds(off, 16)]
                    for f in range(d):
                        fvec = jnp.full((16,), f, jnp.int32)
                        vals = plsc.load_gather(w_v, [fvec, idx])
                        out_v[f, pl.ds(off, 16)] = vals
                    return 0

                lax.fori_loop(0, size // 16, body, 0)
                pltpu.sync_copy(out_v.at[:, pl.ds(0, size)],
                                o_hbm.at[:, pl.ds(base, size)])

            table_pass(wx_v, ox_hbm)
            table_pass(wz_v, oz_hbm)

        @pl.when(wid < nw - 1)
        def _full():
            run(b_full, b_full)

        @pl.when(wid == nw - 1)
        def _last():
            run(r_last, b_last)

    return gather


def kernel(x, edge_attr, W_x, W_z):
    n = x.shape[0]
    d = W_x.shape[1]
    gx, gz = _make_sc_gather(n, d, W_x.shape[0])(W_x.T, W_z.T, x)
    he_t = _bessel_edges_t(edge_attr.T)
    return gx[:, :n].T, gz[:, :n].T, he_t.T


# speedup vs baseline: 1.7469x; 1.0396x over previous
import functools

import jax
import jax.numpy as jnp
from jax import lax
from jax.experimental import pallas as pl
from jax.experimental.pallas import tpu as pltpu
from jax.experimental.pallas import tpu_sc as plsc

_CUTOFF = 5.0
_NUM_BASIS = 16
_BC = 64000

_NC = 2
_NS = 16
_NW = _NC * _NS

_INV_C = 1.0 / _CUTOFF
_SQ2C = 0.6324555320336759
_P0 = 3.14158476
_P1 = -5.16724799
_P2 = 2.54287433
_P3 = -0.55715608


def _bessel_body(e_ref, o_ref):
    e = e_ref[...]
    r2 = jnp.sum(e * e, axis=0, keepdims=True)
    irs = lax.rsqrt(r2)
    scale = _SQ2C * irs
    thpi = _INV_C * (r2 * irs)
    n = (lax.broadcasted_iota(jnp.int32, (_NUM_BASIS, e.shape[1]), 0) + 1
         ).astype(jnp.float32)
    m = n * thpi
    k = jnp.round(m)
    u = m - k
    s = u * u
    p = _P3
    for c in (_P2, _P1, _P0):
        p = p * s + c
    p = p * u
    sb = k.astype(jnp.int32) << 31
    y = lax.bitcast_convert_type(
        lax.bitcast_convert_type(p, jnp.int32) ^ sb, jnp.float32)
    o_ref[...] = y * scale


def _bessel_edges_t(ea_t):
    ne = ea_t.shape[1]
    grid = ne // _BC
    assert grid * _BC == ne
    return pl.pallas_call(
        _bessel_body,
        grid=(grid,),
        in_specs=[pl.BlockSpec((3, _BC), lambda i: (0, i))],
        out_specs=pl.BlockSpec((_NUM_BASIS, _BC), lambda i: (0, i)),
        out_shape=jax.ShapeDtypeStruct((_NUM_BASIS, ne), jnp.float32),
    )(ea_t)


def _make_sc_gather(n, d, species):
    n_pad = ((n + 127) // 128) * 128
    nw = _NS
    b_full = ((-(-n_pad // nw) + 127) // 128) * 128
    b_last = n_pad - (nw - 1) * b_full
    r_last = n - (nw - 1) * b_full
    assert 0 < b_last <= b_full and b_last % 128 == 0
    assert 0 < r_last <= b_last and r_last % 8 == 0 and (b_last - r_last) % 16 == 0
    mesh = plsc.VectorSubcoreMesh(core_axis_name="c", subcore_axis_name="s",
                                  num_cores=1)

    @functools.partial(
        pl.kernel,
        mesh=mesh,
        compiler_params=pltpu.CompilerParams(use_tc_tiling_on_sc=True,
                                            needs_layout_passes=False),
        out_type=(jax.ShapeDtypeStruct((d, n_pad), jnp.float32),
                  jax.ShapeDtypeStruct((d, n_pad), jnp.float32)),
        scratch_types=[
            pltpu.VMEM((d, species), jnp.float32),
            pltpu.VMEM((d, species), jnp.float32),
            pltpu.VMEM((b_full,), jnp.int32),
            pltpu.VMEM((d, b_full), jnp.float32),
        ],
    )
    def gather(wxt_hbm, wzt_hbm, idx_hbm, ox_hbm, oz_hbm,
               wx_v, wz_v, idx_v, out_v):
        wid = lax.axis_index("s")
        base = wid * b_full
        pltpu.sync_copy(wxt_hbm, wx_v)
        pltpu.sync_copy(wzt_hbm, wz_v)

        def run(n_idx, size):
            pltpu.sync_copy(idx_hbm.at[pl.ds(base, n_idx)],
                            idx_v.at[pl.ds(0, n_idx)])
            if n_idx < size:
                zeros = jnp.zeros((16,), jnp.int32)
                for off in range(n_idx, size, 16):
                    idx_v[pl.ds(off, 16)] = zeros

            def table_pass(w_v, o_hbm):
                def body(ci, _):
                    off = ci * 16
                    idx = idx_v[pl.ds(off, 16)]
                    for f in range(d):
                        fvec = jnp.full((16,), f, jnp.int32)
                        vals = plsc.load_gather(w_v, [fvec, idx])
                        out_v[f, pl.ds(off, 16)] = vals
                    return 0

                lax.fori_loop(0, size // 16, body, 0)
                pltpu.sync_copy(out_v.at[:, pl.ds(0, size)],
                                o_hbm.at[:, pl.ds(base, size)])

            table_pass(wx_v, ox_hbm)
            table_pass(wz_v, oz_hbm)

        @pl.when(wid < nw - 1)
        def _full():
            run(b_full, b_full)

        @pl.when(wid == nw - 1)
        def _last():
            run(r_last, b_last)

    return gather


def kernel(x, edge_attr, W_x, W_z):
    n = x.shape[0]
    d = W_x.shape[1]
    gx, gz = _make_sc_gather(n, d, W_x.shape[0])(W_x.T, W_z.T, x)
    he_t = _bessel_edges_t(edge_attr.T)
    return gx[:, :n].T, gz[:, :n].T, he_t.T
